# Initial kernel scaffold; baseline (speedup 1.0000x reference)
import jax
import jax.numpy as jnp
from jax.experimental import pallas as pl


def kernel(x, edge_index, W1n, W1r, b1, W2n, W2r, b2, ln_g, ln_b, Wd, bd):
    raise NotImplementedError("write your pallas kernel here")


# speedup vs baseline: 4.2920x; 4.2920x over previous
import functools

import jax
import jax.numpy as jnp
from jax import lax
from jax.experimental import pallas as pl
from jax.experimental.pallas import tpu as pltpu
from jax.experimental.pallas import tpu_sc as plsc

N = 10000
E = 160000
D = 256
H = 128
NC = 2
NS = 16
C = 128

E_PAD1 = ((E + NS * C - 1) // (NS * C)) * (NS * C)
CHUNKS1 = E_PAD1 // (NS * C)
SP_ROWS = 10240
DUMMY = N

E_PAD3 = ((E + NC * NS * 128 - 1) // (NC * NS * 128)) * (NC * NS * 128)
EPT3 = E_PAD3 // (NC * NS)


@functools.cache
def _make_sc_conv():
    mesh = plsc.VectorSubcoreMesh(core_axis_name="c", subcore_axis_name="s",
                                  num_cores=NC, num_subcores=NS)

    @functools.partial(
        pl.kernel, mesh=mesh,
        out_type=jax.ShapeDtypeStruct((NC * SP_ROWS, H), jnp.float32),
        scratch_types=[
            pltpu.VMEM_SHARED((SP_ROWS, H), jnp.float32),
            pltpu.VMEM((C, H), jnp.float32),
            pltpu.VMEM((C,), jnp.int32),
            pltpu.VMEM((C,), jnp.int32),
            pltpu.SemaphoreType.DMA,
        ],
    )
    def conv(table, src, dst, zrows, agg_out, agg_s, buf, sidx, didx, sem):
        cid = lax.axis_index("c")
        sid = lax.axis_index("s")

        pltpu.sync_copy(zrows, buf)
        for k in range(SP_ROWS // (NS * C)):
            r0 = sid * (SP_ROWS // NS) + k * C
            pltpu.sync_copy(buf, agg_s.at[pl.ds(r0, C)])
        plsc.subcore_barrier()

        ept = E_PAD1 // NS
        half_off = cid * N

        def chunk(k, _):
            base = sid * ept + k * C
            pltpu.sync_copy(src.at[pl.ds(base, C)], sidx)
            pltpu.sync_copy(dst.at[pl.ds(base, C)], didx)
            for j in range(C // 16):
                sl = pl.ds(j * 16, 16)
                sidx[sl] = sidx[sl] + half_off
            pltpu.async_copy(table.at[sidx], buf, sem).wait()
            pltpu.sync_copy(buf, agg_s.at[didx], add=True)
            return 0

        lax.fori_loop(0, CHUNKS1, chunk, 0)
        plsc.subcore_barrier()

        for k in range(SP_ROWS // (NS * C)):
            r0 = sid * (SP_ROWS // NS) + k * C
            pltpu.sync_copy(agg_s.at[pl.ds(r0, C)], buf)
            pltpu.sync_copy(buf, agg_out.at[pl.ds(cid * SP_ROWS + r0, C)])

    return conv


@functools.cache
def _make_sc_deg():
    mesh = plsc.VectorSubcoreMesh(core_axis_name="c", subcore_axis_name="s",
                                  num_cores=NC, num_subcores=NS)

    @functools.partial(
        pl.kernel, mesh=mesh,
        out_type=jax.ShapeDtypeStruct((NC * SP_ROWS, H), jnp.float32),
        scratch_types=[
            pltpu.VMEM_SHARED((SP_ROWS, H), jnp.float32),
            pltpu.VMEM((C, H), jnp.float32),
            pltpu.VMEM((C, H), jnp.float32),
            pltpu.VMEM((C,), jnp.int32),
        ],
    )
    def deg(dst, zrows, orows, deg_out, deg_s, o128, buf, didx):
        cid = lax.axis_index("c")
        sid = lax.axis_index("s")
        pltpu.sync_copy(orows, o128)
        pltpu.sync_copy(zrows, buf)
        for k in range(SP_ROWS // (NS * C)):
            r0 = sid * (SP_ROWS // NS) + k * C
            pltpu.sync_copy(buf, deg_s.at[pl.ds(r0, C)])
        plsc.subcore_barrier()

        ept = E_PAD3 // (NC * NS)

        def chunk(k, _):
            base = (cid * NS + sid) * ept + k * C
            pltpu.sync_copy(dst.at[pl.ds(base, C)], didx)
            pltpu.sync_copy(o128, deg_s.at[didx], add=True)
            return 0

        lax.fori_loop(0, ept // C, chunk, 0)
        plsc.subcore_barrier()

        for k in range(SP_ROWS // (NS * C)):
            r0 = sid * (SP_ROWS // NS) + k * C
            pltpu.sync_copy(deg_s.at[pl.ds(r0, C)], buf)
            pltpu.sync_copy(buf, deg_out.at[pl.ds(cid * SP_ROWS + r0, C)])

    return deg


def _tc1_body(agg, dega, degb, x, w1n, w1r, b1, out):
    scale = 1.0 / jnp.maximum(dega[:] + degb[:], 1.0)
    h = (jnp.dot(agg[:] * scale, w1n[:], preferred_element_type=jnp.float32)
         + jnp.dot(x[:], w1r[:], preferred_element_type=jnp.float32)
         + b1[:])
    out[:] = jnp.maximum(h, 0.0)


def _tc2_body(agg, dega, degb, h1, w2n, w2r, b2, g, b, wda, wdb, bd,
              h_out, s_out, t_out):
    scale = 1.0 / jnp.maximum(dega[:] + degb[:], 1.0)
    h2 = (jnp.dot(agg[:] * scale, w2n[:], preferred_element_type=jnp.float32)
          + jnp.dot(h1[:], w2r[:], preferred_element_type=jnp.float32)
          + b2[:])
    h2 = jnp.maximum(h2, 0.0)
    mu = jnp.mean(h2, axis=-1, keepdims=True)
    var = jnp.mean((h2 - mu) ** 2, axis=-1, keepdims=True)
    hn = (h2 - mu) * jax.lax.rsqrt(var + 1e-5) * g[:] + b[:]
    h_out[:] = hn
    s_out[:] = jnp.dot(hn, wda[:], preferred_element_type=jnp.float32) + bd[:]
    t_out[:] = jnp.dot(hn, wdb[:], preferred_element_type=jnp.float32)


@functools.cache
def _make_sc_decode():
    mesh = plsc.VectorSubcoreMesh(core_axis_name="c", subcore_axis_name="s",
                                  num_cores=NC, num_subcores=NS)

    @functools.partial(
        pl.kernel, mesh=mesh,
        out_type=jax.ShapeDtypeStruct((E_PAD3,), jnp.float32),
        compiler_params=pltpu.CompilerParams(needs_layout_passes=False),
        scratch_types=[
            pltpu.VMEM((SP_ROWS,), jnp.float32),
            pltpu.VMEM((SP_ROWS,), jnp.float32),
            pltpu.VMEM((EPT3,), jnp.int32),
            pltpu.VMEM((EPT3,), jnp.int32),
            pltpu.VMEM((EPT3,), jnp.float32),
        ],
    )
    def decode(s_hbm, t_hbm, src, dst, out, sv, tv, si, di, ov):
        cid = lax.axis_index("c")
        sid = lax.axis_index("s")
        wid = cid * NS + sid
        base = wid * EPT3
        pltpu.sync_copy(s_hbm, sv)
        pltpu.sync_copy(t_hbm, tv)
        pltpu.sync_copy(src.at[pl.ds(base, EPT3)], si)
        pltpu.sync_copy(dst.at[pl.ds(base, EPT3)], di)

        def step(i, _):
            sl = pl.ds(i * 16, 16)
            a = plsc.load_gather(sv, [si[sl]])
            c = plsc.load_gather(tv, [di[sl]])
            ov[sl] = 1.0 / (1.0 + jnp.exp(-(a + c)))
            return 0

        lax.fori_loop(0, EPT3 // 16, step, 0)
        pltpu.sync_copy(ov, out.at[pl.ds(base, EPT3)])

    return decode


def kernel(x, edge_index, W1n, W1r, b1, W2n, W2r, b2, ln_g, ln_b, Wd, bd):
    src = edge_index[0]
    dst = edge_index[1]
    pad1 = E_PAD1 - E
    src1 = jnp.concatenate([src, jnp.zeros((pad1,), jnp.int32)])
    dst1 = jnp.concatenate([dst, jnp.full((pad1,), DUMMY, jnp.int32)])
    pad3 = E_PAD3 - E
    src3 = jnp.concatenate([src, jnp.zeros((pad3,), jnp.int32)])
    dst3 = jnp.concatenate([dst, jnp.zeros((pad3,), jnp.int32)])
    dstd = jnp.concatenate([dst, jnp.full((pad3,), DUMMY, jnp.int32)])

    zrows = jnp.zeros((C, H), jnp.float32)
    orows = jnp.ones((C, H), jnp.float32)

    xcat = jnp.concatenate([x[:, :H], x[:, H:]], axis=0)
    degc = _make_sc_deg()(dstd, zrows, orows)
    dega = degc[:N, :1]
    degb = degc[SP_ROWS:SP_ROWS + N, :1]
    agg1c = _make_sc_conv()(xcat, src1, dst1, zrows)
    agg1 = jnp.concatenate([agg1c[:N], agg1c[SP_ROWS:SP_ROWS + N]], axis=1)

    B = 2000
    NB = N // B
    row = pl.BlockSpec((B, D), lambda i: (i, 0))
    col1 = pl.BlockSpec((B, 1), lambda i: (i, 0))
    wfull = pl.BlockSpec((D, D), lambda i: (0, 0))
    vfull = pl.BlockSpec((D,), lambda i: (0,))

    h1 = pl.pallas_call(
        _tc1_body,
        grid=(NB,),
        in_specs=[row, col1, col1, row, wfull, wfull, vfull],
        out_specs=row,
        out_shape=jax.ShapeDtypeStruct((N, D), jnp.float32),
    )(agg1, dega, degb, x, W1n, W1r, b1)

    h1cat = jnp.concatenate([h1[:, :H], h1[:, H:]], axis=0)
    agg2c = _make_sc_conv()(h1cat, src1, dst1, zrows)
    agg2 = jnp.concatenate([agg2c[:N], agg2c[SP_ROWS:SP_ROWS + N]], axis=1)

    wcol = pl.BlockSpec((D, 1), lambda i: (0, 0))
    h, s, t = pl.pallas_call(
        _tc2_body,
        grid=(NB,),
        in_specs=[row, col1, col1, row, wfull, wfull, vfull, vfull, vfull,
                  wcol, wcol, pl.BlockSpec((1,), lambda i: (0,))],
        out_specs=[row, col1, col1],
        out_shape=[
            jax.ShapeDtypeStruct((N, D), jnp.float32),
            jax.ShapeDtypeStruct((N, 1), jnp.float32),
            jax.ShapeDtypeStruct((N, 1), jnp.float32),
        ],
    )(agg2, dega, degb, h1, W2n, W2r, b2, ln_g, ln_b, Wd[:D], Wd[D:], bd)

    sp = jnp.pad(s.reshape(N), (0, SP_ROWS - N))
    tp = jnp.pad(t.reshape(N), (0, SP_ROWS - N))
    newp = _make_sc_decode()(sp, tp, src3, dst3)
    new_h = newp[:E, None]
    return (h, new_h)
